# Initial kernel scaffold; baseline (speedup 1.0000x reference)
import jax
import jax.numpy as jnp
from jax.experimental import pallas as pl


def kernel(x, A_indices, A_values, tuplefeat, x_tables, ea_tables, tuple_table):
    raise NotImplementedError("write your pallas kernel here")


# speedup vs baseline: 2.3642x; 2.3642x over previous
import functools

import jax
import jax.numpy as jnp
from jax import lax
from jax.experimental import pallas as pl
from jax.experimental.pallas import tpu as pltpu
from jax.experimental.pallas import tpu_sc as plsc

N = 10000
E = 320000
D = 128


def _comb_body(ea_flat_ref, out_ref):
    k = lax.broadcasted_iota(jnp.int32, (1024, 1), 0)
    d0 = k // 100
    d1 = (k // 10) % 10
    d2 = k % 10
    col = lax.broadcasted_iota(jnp.int32, (1024, 32), 1)
    valid = k < 1000
    mh = (((col == d0) | (col == 10 + d1) | (col == 20 + d2)) & valid)
    out_ref[...] = jnp.dot(mh.astype(jnp.float32), ea_flat_ref[...],
                           preferred_element_type=jnp.float32)


def _build_comb(ea_flat):
    return pl.pallas_call(
        _comb_body,
        out_shape=jax.ShapeDtypeStruct((1024, D), jnp.float32),
    )(ea_flat)


def _keys_body(a0_ref, a1_ref, a2_ref, out_ref):
    out_ref[...] = a0_ref[...] * 100 + a1_ref[...] * 10 + a2_ref[...]


def _build_keys(a0, a1, a2):
    return pl.pallas_call(
        _keys_body,
        out_shape=jax.ShapeDtypeStruct(a0.shape, jnp.int32),
    )(a0, a1, a2)


def _x_body(xp_ref, tab_ref, out_ref):
    xv = xp_ref[...]
    col = lax.broadcasted_iota(jnp.int32, (400, 1024), 1)
    mh = jnp.zeros((400, 1024), jnp.float32)
    for c in range(16):
        off = 100 * c if c < 9 else 900
        mh = mh + (col == (xv[:, c:c + 1] + off)).astype(jnp.float32)
    out_ref[...] = jnp.dot(mh, tab_ref[...],
                           preferred_element_type=jnp.float32)


def _x_encode(xp, xt_flat):
    return pl.pallas_call(
        _x_body,
        grid=(N // 400,),
        in_specs=[
            pl.BlockSpec((400, 16), lambda i: (i, 0)),
            pl.BlockSpec((1024, D), lambda i: (0, 0)),
        ],
        out_specs=pl.BlockSpec((400, D), lambda i: (i, 0)),
        out_shape=jax.ShapeDtypeStruct((N, D), jnp.float32),
    )(xp, xt_flat)


_info = plsc.get_sparse_core_info()
_NC, _NS = _info.num_cores, _info.num_subcores
_NW = _NC * _NS
_RPW = E // _NW
_C = 80
_NCHUNK = _RPW // _C

_sc_mesh = plsc.VectorSubcoreMesh(core_axis_name="c", subcore_axis_name="s")


@functools.partial(
    pl.kernel,
    mesh=_sc_mesh,
    out_type=[
        jax.ShapeDtypeStruct((E, D), jnp.float32),
        jax.ShapeDtypeStruct((E, D), jnp.float32),
    ],
    scratch_types=[
        pltpu.VMEM((_C,), jnp.int32),
        pltpu.VMEM((_C, D), jnp.float32),
        pltpu.SemaphoreType.DMA,
    ],
)
def _sc_gather(keys_hbm, comb_hbm, tf_hbm, tt_hbm, ea_out, tup_out,
               idx_v, rows_v, sem):
    wid = lax.axis_index("s") * _NC + lax.axis_index("c")
    base0 = wid * _RPW

    def body(i, carry):
        base = base0 + i * _C
        pltpu.sync_copy(keys_hbm.at[pl.ds(base, _C)], idx_v)
        pltpu.async_copy(comb_hbm.at[idx_v], rows_v, sem).wait()
        pltpu.sync_copy(rows_v, ea_out.at[pl.ds(base, _C), :])
        pltpu.sync_copy(tf_hbm.at[pl.ds(base, _C)], idx_v)
        pltpu.async_copy(tt_hbm.at[idx_v], rows_v, sem).wait()
        pltpu.sync_copy(rows_v, tup_out.at[pl.ds(base, _C), :])
        return carry

    lax.fori_loop(0, _NCHUNK, body, 0)


def kernel(x, A_indices, A_values, tuplefeat, x_tables, ea_tables, tuple_table):
    del A_indices
    ea_flat = jnp.concatenate(
        [ea_tables.reshape(30, D), jnp.zeros((2, D), jnp.float32)], axis=0)
    comb = _build_comb(ea_flat)
    a0 = A_values[:, 0].reshape(E // 512, 512)
    a1 = A_values[:, 1].reshape(E // 512, 512)
    a2 = A_values[:, 2].reshape(E // 512, 512)
    keys = _build_keys(a0, a1, a2).reshape(E)
    xp = jnp.pad(x, ((0, 0), (0, 7)))
    xt_flat = jnp.concatenate(
        [x_tables.reshape(900, D), jnp.zeros((124, D), jnp.float32)], axis=0)
    x_emb = _x_encode(xp, xt_flat)
    ea_emb, tuple_emb = _sc_gather(keys, comb, tuplefeat, tuple_table)
    return (x_emb, ea_emb, tuple_emb)


# speedup vs baseline: 2.6630x; 1.1264x over previous
import functools

import jax
import jax.numpy as jnp
from jax import lax
from jax.experimental import pallas as pl
from jax.experimental.pallas import tpu as pltpu
from jax.experimental.pallas import tpu_sc as plsc

N = 10000
E = 320000
D = 128


def _comb_body(ea_flat_ref, out_ref):
    k = lax.broadcasted_iota(jnp.int32, (1024, 1), 0)
    d0 = k // 100
    d1 = (k // 10) % 10
    d2 = k % 10
    col = lax.broadcasted_iota(jnp.int32, (1024, 32), 1)
    valid = k < 1000
    mh = (((col == d0) | (col == 10 + d1) | (col == 20 + d2)) & valid)
    out_ref[...] = jnp.dot(mh.astype(jnp.float32), ea_flat_ref[...],
                           preferred_element_type=jnp.float32)


def _build_comb(ea_flat):
    return pl.pallas_call(
        _comb_body,
        out_shape=jax.ShapeDtypeStruct((1024, D), jnp.float32),
    )(ea_flat)


def _keys_body(a0_ref, a1_ref, a2_ref, out_ref):
    out_ref[...] = a0_ref[...] * 100 + a1_ref[...] * 10 + a2_ref[...]


def _build_keys(a0, a1, a2):
    return pl.pallas_call(
        _keys_body,
        out_shape=jax.ShapeDtypeStruct(a0.shape, jnp.int32),
    )(a0, a1, a2)


def _x_body(xp_ref, tab_ref, out_ref):
    xv = xp_ref[...]
    col = lax.broadcasted_iota(jnp.int32, (400, 1024), 1)
    mh = jnp.zeros((400, 1024), jnp.float32)
    for c in range(16):
        off = 100 * c if c < 9 else 900
        mh = mh + (col == (xv[:, c:c + 1] + off)).astype(jnp.float32)
    out_ref[...] = jnp.dot(mh, tab_ref[...],
                           preferred_element_type=jnp.float32)


def _x_encode(xp, xt_flat):
    return pl.pallas_call(
        _x_body,
        grid=(N // 400,),
        in_specs=[
            pl.BlockSpec((400, 16), lambda i: (i, 0)),
            pl.BlockSpec((1024, D), lambda i: (0, 0)),
        ],
        out_specs=pl.BlockSpec((400, D), lambda i: (i, 0)),
        out_shape=jax.ShapeDtypeStruct((N, D), jnp.float32),
    )(xp, xt_flat)


_info = plsc.get_sparse_core_info()
_NC, _NS = _info.num_cores, _info.num_subcores
_NW = _NC * _NS
_RPW = E // _NW
_C = 80
_NB = 5
_NGRP = _RPW // (_C * _NB)

_sc_mesh = plsc.VectorSubcoreMesh(core_axis_name="c", subcore_axis_name="s")


@functools.partial(
    pl.kernel,
    mesh=_sc_mesh,
    out_type=[
        jax.ShapeDtypeStruct((E, D), jnp.float32),
        jax.ShapeDtypeStruct((E, D), jnp.float32),
    ],
    scratch_types=[
        pltpu.VMEM((_RPW,), jnp.int32),
        [pltpu.VMEM((_C, D), jnp.float32)] * _NB,
        pltpu.SemaphoreType.DMA,
        pltpu.SemaphoreType.DMA,
    ],
)
def _sc_gather(keys_hbm, comb_hbm, tf_hbm, tt_hbm, ea_out, tup_out,
               idx_v, rows, gsem, osem):
    wid = lax.axis_index("s") * _NC + lax.axis_index("c")
    base0 = wid * _RPW

    def run_job(idx_hbm, table_hbm, out_hbm):
        pltpu.sync_copy(idx_hbm.at[pl.ds(base0, _RPW)], idx_v)

        def group(j, carry):
            goff = j * (_C * _NB)

            @pl.when(j > 0)
            def _():
                for b in range(_NB):
                    pltpu.make_async_copy(
                        rows[b], out_hbm.at[pl.ds(base0, _C), :], osem
                    ).wait()

            for b in range(_NB):
                pltpu.async_copy(
                    table_hbm.at[idx_v.at[pl.ds(goff + b * _C, _C)]],
                    rows[b], gsem)
            for b in range(_NB):
                pltpu.make_async_copy(
                    table_hbm.at[idx_v.at[pl.ds(goff + b * _C, _C)]],
                    rows[b], gsem).wait()
                pltpu.async_copy(
                    rows[b], out_hbm.at[pl.ds(base0 + goff + b * _C, _C), :],
                    osem)
            return carry

        lax.fori_loop(0, _NGRP, group, 0)
        for b in range(_NB):
            pltpu.make_async_copy(
                rows[b], out_hbm.at[pl.ds(base0, _C), :], osem).wait()

    run_job(keys_hbm, comb_hbm, ea_out)
    run_job(tf_hbm, tt_hbm, tup_out)


def kernel(x, A_indices, A_values, tuplefeat, x_tables, ea_tables, tuple_table):
    del A_indices
    ea_flat = jnp.concatenate(
        [ea_tables.reshape(30, D), jnp.zeros((2, D), jnp.float32)], axis=0)
    comb = _build_comb(ea_flat)
    a0 = A_values[:, 0].reshape(E // 512, 512)
    a1 = A_values[:, 1].reshape(E // 512, 512)
    a2 = A_values[:, 2].reshape(E // 512, 512)
    keys = _build_keys(a0, a1, a2).reshape(E)
    xp = jnp.pad(x, ((0, 0), (0, 7)))
    xt_flat = jnp.concatenate(
        [x_tables.reshape(900, D), jnp.zeros((124, D), jnp.float32)], axis=0)
    x_emb = _x_encode(xp, xt_flat)
    ea_emb, tuple_emb = _sc_gather(keys, comb, tuplefeat, tuple_table)
    return (x_emb, ea_emb, tuple_emb)


# speedup vs baseline: 6.1354x; 2.3040x over previous
import functools

import jax
import jax.numpy as jnp
from jax import lax
from jax.experimental import pallas as pl
from jax.experimental.pallas import tpu as pltpu
from jax.experimental.pallas import tpu_sc as plsc

N = 10000
E = 320000
D = 128

_info = plsc.get_sparse_core_info()
_NC, _NS = _info.num_cores, _info.num_subcores
_NW = _NC * _NS
_RPW_ROWS = E // _NW
_C = 128
_NB = 6
_NFULL = _RPW_ROWS // _C
_NGRP = _NFULL // _NB
_TAIL = _RPW_ROWS - _NFULL * _C

_COMB_REP = 1024
_TUP_REP = 32


def _comb_body(ea_flat_ref, out_ref):
    k = lax.broadcasted_iota(jnp.int32, (_COMB_REP, 1), 0)
    d0 = k // 100
    d1 = (k // 10) % 10
    d2 = k % 10
    col = lax.broadcasted_iota(jnp.int32, (_COMB_REP, 32), 1)
    valid = k < 1000
    mh = (((col == d0) | (col == 10 + d1) | (col == 20 + d2)) & valid)
    out_ref[0] = jnp.dot(mh.astype(jnp.float32), ea_flat_ref[...],
                         preferred_element_type=jnp.float32)


def _build_comb(ea_flat):
    return pl.pallas_call(
        _comb_body,
        grid=(_NW,),
        in_specs=[pl.BlockSpec((32, D), lambda i: (0, 0))],
        out_specs=pl.BlockSpec((1, _COMB_REP, D), lambda i: (i, 0, 0)),
        out_shape=jax.ShapeDtypeStruct((_NW, _COMB_REP, D), jnp.float32),
    )(ea_flat)


def _tup_body(tab_ref, out_ref):
    k = lax.broadcasted_iota(jnp.int32, (_TUP_REP, 1), 0)
    col = lax.broadcasted_iota(jnp.int32, (_TUP_REP, 32), 1)
    mh = ((col == k) & (k < 20) & (col < 20))
    out_ref[0] = jnp.dot(mh.astype(jnp.float32), tab_ref[...],
                         preferred_element_type=jnp.float32)


def _build_tup(tab_pad):
    return pl.pallas_call(
        _tup_body,
        grid=(_NW,),
        in_specs=[pl.BlockSpec((32, D), lambda i: (0, 0))],
        out_specs=pl.BlockSpec((1, _TUP_REP, D), lambda i: (i, 0, 0)),
        out_shape=jax.ShapeDtypeStruct((_NW, _TUP_REP, D), jnp.float32),
    )(tab_pad)


def _keys_body(a0_ref, a1_ref, a2_ref, out_ref):
    r = lax.broadcasted_iota(jnp.int32, a0_ref.shape, 0)
    c = lax.broadcasted_iota(jnp.int32, a0_ref.shape, 1)
    wid = (r * 512 + c) // _RPW_ROWS
    out_ref[...] = (a0_ref[...] * 100 + a1_ref[...] * 10 + a2_ref[...]
                    + wid * _COMB_REP)


def _build_keys(a0, a1, a2):
    return pl.pallas_call(
        _keys_body,
        out_shape=jax.ShapeDtypeStruct(a0.shape, jnp.int32),
    )(a0, a1, a2)


def _tkeys_body(t_ref, out_ref):
    r = lax.broadcasted_iota(jnp.int32, t_ref.shape, 0)
    c = lax.broadcasted_iota(jnp.int32, t_ref.shape, 1)
    wid = (r * 512 + c) // _RPW_ROWS
    out_ref[...] = t_ref[...] + wid * _TUP_REP


def _build_tkeys(t2d):
    return pl.pallas_call(
        _tkeys_body,
        out_shape=jax.ShapeDtypeStruct(t2d.shape, jnp.int32),
    )(t2d)


def _x_body(xp_ref, tab_ref, out_ref):
    xv = xp_ref[...]
    col = lax.broadcasted_iota(jnp.int32, (400, 1024), 1)
    mh = jnp.zeros((400, 1024), jnp.float32)
    for c in range(16):
        off = 100 * c if c < 9 else 900
        mh = mh + (col == (xv[:, c:c + 1] + off)).astype(jnp.float32)
    out_ref[...] = jnp.dot(mh, tab_ref[...],
                           preferred_element_type=jnp.float32)


def _x_encode(xp, xt_flat):
    return pl.pallas_call(
        _x_body,
        grid=(N // 400,),
        in_specs=[
            pl.BlockSpec((400, 16), lambda i: (i, 0)),
            pl.BlockSpec((1024, D), lambda i: (0, 0)),
        ],
        out_specs=pl.BlockSpec((400, D), lambda i: (i, 0)),
        out_shape=jax.ShapeDtypeStruct((N, D), jnp.float32),
    )(xp, xt_flat)


_sc_mesh = plsc.VectorSubcoreMesh(core_axis_name="c", subcore_axis_name="s")


@functools.partial(
    pl.kernel,
    mesh=_sc_mesh,
    out_type=[
        jax.ShapeDtypeStruct((E, D), jnp.float32),
        jax.ShapeDtypeStruct((E, D), jnp.float32),
    ],
    scratch_types=[
        pltpu.VMEM((_RPW_ROWS,), jnp.int32),
        [pltpu.VMEM((_C, D), jnp.float32)] * _NB,
        pltpu.SemaphoreType.DMA,
        pltpu.SemaphoreType.DMA,
    ],
)
def _sc_gather(keys_hbm, comb_hbm, tf_hbm, tt_hbm, ea_out, tup_out,
               idx_v, rows, gsem, osem):
    wid = lax.axis_index("s") * _NC + lax.axis_index("c")
    base0 = wid * _RPW_ROWS

    def run_job(idx_hbm, table_hbm, out_hbm):
        pltpu.sync_copy(idx_hbm.at[pl.ds(base0, _RPW_ROWS)], idx_v)

        def fire(c, b):
            pltpu.async_copy(
                table_hbm.at[idx_v.at[pl.ds(c * _C, _C)]], rows[b], gsem)

        def wait_fire_out(c, b):
            pltpu.make_async_copy(
                table_hbm.at[idx_v.at[pl.ds(c * _C, _C)]], rows[b],
                gsem).wait()
            pltpu.async_copy(
                rows[b], out_hbm.at[pl.ds(base0 + c * _C, _C), :], osem)

        def drain_out(b):
            pltpu.make_async_copy(
                rows[b], out_hbm.at[pl.ds(base0, _C), :], osem).wait()

        def group(j, carry):
            @pl.when(j > 0)
            def _():
                for b in range(_NB):
                    drain_out(b)
            for b in range(_NB):
                fire(j * _NB + b, b)
            for b in range(_NB):
                wait_fire_out(j * _NB + b, b)
            return carry

        lax.fori_loop(0, _NGRP, group, 0)
        for b in range(_NB):
            drain_out(b)
        toff = base0 + _NFULL * _C
        tail_rows = rows[0].at[pl.ds(0, _TAIL), :]
        pltpu.async_copy(
            table_hbm.at[idx_v.at[pl.ds(_NFULL * _C, _TAIL)]],
            tail_rows, gsem)
        pltpu.make_async_copy(
            table_hbm.at[idx_v.at[pl.ds(_NFULL * _C, _TAIL)]],
            tail_rows, gsem).wait()
        pltpu.sync_copy(tail_rows, out_hbm.at[pl.ds(toff, _TAIL), :])

    run_job(keys_hbm, comb_hbm, ea_out)
    run_job(tf_hbm, tt_hbm, tup_out)


def kernel(x, A_indices, A_values, tuplefeat, x_tables, ea_tables, tuple_table):
    del A_indices
    ea_flat = jnp.concatenate(
        [ea_tables.reshape(30, D), jnp.zeros((2, D), jnp.float32)], axis=0)
    comb = _build_comb(ea_flat).reshape(_NW * _COMB_REP, D)
    tup_rep = _build_tup(
        jnp.concatenate([tuple_table, jnp.zeros((12, D), jnp.float32)],
                        axis=0)).reshape(_NW * _TUP_REP, D)
    a0 = A_values[:, 0].reshape(E // 512, 512)
    a1 = A_values[:, 1].reshape(E // 512, 512)
    a2 = A_values[:, 2].reshape(E // 512, 512)
    keys = _build_keys(a0, a1, a2).reshape(E)
    tkeys = _build_tkeys(tuplefeat.reshape(E // 512, 512)).reshape(E)
    xp = jnp.pad(x, ((0, 0), (0, 7)))
    xt_flat = jnp.concatenate(
        [x_tables.reshape(900, D), jnp.zeros((124, D), jnp.float32)], axis=0)
    x_emb = _x_encode(xp, xt_flat)
    ea_emb, tuple_emb = _sc_gather(keys, comb, tkeys, tup_rep)
    return (x_emb, ea_emb, tuple_emb)


# speedup vs baseline: 6.5132x; 1.0616x over previous
import functools

import jax
import jax.numpy as jnp
from jax import lax
from jax.experimental import pallas as pl
from jax.experimental.pallas import tpu as pltpu
from jax.experimental.pallas import tpu_sc as plsc

N = 10000
E = 320000
D = 128

_info = plsc.get_sparse_core_info()
_NC, _NS = _info.num_cores, _info.num_subcores
_NW = _NC * _NS
_RPW = E // _NW
_C = 128
_NB = 6
_NFULL = _RPW // _C
_NGRP = _NFULL // _NB
_TAIL = _RPW - _NFULL * _C

_COMB_REP = 1024
_TUP_REP = 32


def _tables_body(tabs_ref, out_ref):
    pid = pl.program_id(0)
    k = lax.broadcasted_iota(jnp.int32, (1024, 1), 0)
    col = lax.broadcasted_iota(jnp.int32, (1024, 64), 1)
    mh_comb = (((col == k // 100) | (col == 10 + (k // 10) % 10)
                | (col == 20 + k % 10)) & (k < 1000))
    mh_tup = (col == 32 + k % 32)
    mh = jnp.where(pid < _NW, mh_comb.astype(jnp.float32),
                   mh_tup.astype(jnp.float32))
    out_ref[0] = jnp.dot(mh, tabs_ref[...],
                         preferred_element_type=jnp.float32)


def _build_tables(tabs):
    return pl.pallas_call(
        _tables_body,
        grid=(_NW + 1,),
        in_specs=[pl.BlockSpec((64, D), lambda i: (0, 0))],
        out_specs=pl.BlockSpec((1, 1024, D), lambda i: (i, 0, 0)),
        out_shape=jax.ShapeDtypeStruct((_NW + 1, 1024, D), jnp.float32),
    )(tabs)


def _keys_body(a0_ref, a1_ref, a2_ref, t_ref, k_ref, tk_ref):
    r = lax.broadcasted_iota(jnp.int32, a0_ref.shape, 0)
    c = lax.broadcasted_iota(jnp.int32, a0_ref.shape, 1)
    wid = (r * 512 + c) // _RPW
    k_ref[...] = (a0_ref[...] * 100 + a1_ref[...] * 10 + a2_ref[...]
                  + wid * _COMB_REP)
    tk_ref[...] = t_ref[...] + wid * _TUP_REP


def _build_keys(a0, a1, a2, t2d):
    return pl.pallas_call(
        _keys_body,
        out_shape=[jax.ShapeDtypeStruct(a0.shape, jnp.int32),
                   jax.ShapeDtypeStruct(a0.shape, jnp.int32)],
    )(a0, a1, a2, t2d)


def _x_body(xp_ref, tab_ref, out_ref):
    xv = xp_ref[...]
    col = lax.broadcasted_iota(jnp.int32, (400, 1024), 1)
    mh = jnp.zeros((400, 1024), jnp.float32)
    for c in range(16):
        off = 100 * c if c < 9 else 900
        mh = mh + (col == (xv[:, c:c + 1] + off)).astype(jnp.float32)
    out_ref[...] = jnp.dot(mh, tab_ref[...],
                           preferred_element_type=jnp.float32)


def _x_encode(xp, xt_flat):
    return pl.pallas_call(
        _x_body,
        grid=(N // 400,),
        in_specs=[
            pl.BlockSpec((400, 16), lambda i: (i, 0)),
            pl.BlockSpec((1024, D), lambda i: (0, 0)),
        ],
        out_specs=pl.BlockSpec((400, D), lambda i: (i, 0)),
        out_shape=jax.ShapeDtypeStruct((N, D), jnp.float32),
    )(xp, xt_flat)


_sc_mesh = plsc.VectorSubcoreMesh(core_axis_name="c", subcore_axis_name="s")


@functools.partial(
    pl.kernel,
    mesh=_sc_mesh,
    out_type=[
        jax.ShapeDtypeStruct((E, D), jnp.float32),
        jax.ShapeDtypeStruct((E, D), jnp.float32),
    ],
    scratch_types=[
        pltpu.VMEM((_RPW,), jnp.int32),
        [pltpu.VMEM((_C, D), jnp.float32)] * _NB,
        pltpu.SemaphoreType.DMA,
        pltpu.SemaphoreType.DMA,
    ],
)
def _sc_gather(keys_hbm, comb_hbm, tf_hbm, tt_hbm, ea_out, tup_out,
               idx_v, rows, gsem, osem):
    wid = lax.axis_index("s") * _NC + lax.axis_index("c")
    base0 = wid * _RPW

    def run_job(idx_hbm, table_hbm, out_hbm):
        pltpu.sync_copy(idx_hbm.at[pl.ds(base0, _RPW)], idx_v)

        def fire(c, b):
            pltpu.async_copy(
                table_hbm.at[idx_v.at[pl.ds(c * _C, _C)]], rows[b], gsem)

        def wait_fire_out(c, b):
            pltpu.make_async_copy(
                table_hbm.at[idx_v.at[pl.ds(c * _C, _C)]], rows[b],
                gsem).wait()
            pltpu.async_copy(
                rows[b], out_hbm.at[pl.ds(base0 + c * _C, _C), :], osem)

        def drain_out(b):
            pltpu.make_async_copy(
                rows[b], out_hbm.at[pl.ds(base0, _C), :], osem).wait()

        def group(j, carry):
            for b in range(_NB):
                @pl.when(j > 0)
                def _():
                    drain_out(b)
                fire(j * _NB + b, b)
            for b in range(_NB):
                wait_fire_out(j * _NB + b, b)
            return carry

        lax.fori_loop(0, _NGRP, group, 0)
        for b in range(_NB):
            drain_out(b)
        toff = base0 + _NFULL * _C
        tail_rows = rows[0].at[pl.ds(0, _TAIL), :]
        pltpu.async_copy(
            table_hbm.at[idx_v.at[pl.ds(_NFULL * _C, _TAIL)]],
            tail_rows, gsem)
        pltpu.make_async_copy(
            table_hbm.at[idx_v.at[pl.ds(_NFULL * _C, _TAIL)]],
            tail_rows, gsem).wait()
        pltpu.sync_copy(tail_rows, out_hbm.at[pl.ds(toff, _TAIL), :])

    run_job(keys_hbm, comb_hbm, ea_out)
    run_job(tf_hbm, tt_hbm, tup_out)


def kernel(x, A_indices, A_values, tuplefeat, x_tables, ea_tables, tuple_table):
    del A_indices
    tabs = jnp.concatenate([
        ea_tables.reshape(30, D), jnp.zeros((2, D), jnp.float32),
        tuple_table, jnp.zeros((12, D), jnp.float32)], axis=0)
    tables = _build_tables(tabs)
    comb = tables[:_NW].reshape(_NW * _COMB_REP, D)
    tup_rep = tables[_NW]
    a0 = A_values[:, 0].reshape(E // 512, 512)
    a1 = A_values[:, 1].reshape(E // 512, 512)
    a2 = A_values[:, 2].reshape(E // 512, 512)
    keys, tkeys = _build_keys(a0, a1, a2, tuplefeat.reshape(E // 512, 512))
    ea_emb, tuple_emb = _sc_gather(keys.reshape(E), comb,
                                   tkeys.reshape(E), tup_rep)
    xp = jnp.pad(x, ((0, 0), (0, 7)))
    xt_flat = jnp.concatenate(
        [x_tables.reshape(900, D), jnp.zeros((124, D), jnp.float32)], axis=0)
    x_emb = _x_encode(xp, xt_flat)
    return (x_emb, ea_emb, tuple_emb)


# speedup vs baseline: 16.3452x; 2.5096x over previous
import functools

import jax
import jax.numpy as jnp
from jax import lax
from jax.experimental import pallas as pl
from jax.experimental.pallas import tpu as pltpu
from jax.experimental.pallas import tpu_sc as plsc

N = 10000
E = 320000
D = 128

_info = plsc.get_sparse_core_info()
_NC, _NS = _info.num_cores, _info.num_subcores
_NW = _NC * _NS
_RPW = E // _NW
_C = 128
_NB = 6
_NFULL = _RPW // _C
_NGRP = _NFULL // _NB
_TAIL = _RPW - _NFULL * _C


def _tables_body(tabs_ref, out_ref):
    pid = pl.program_id(0)
    k = lax.broadcasted_iota(jnp.int32, (1024, 1), 0)
    col = lax.broadcasted_iota(jnp.int32, (1024, 64), 1)
    mh_comb = (((col == k // 100) | (col == 10 + (k // 10) % 10)
                | (col == 20 + k % 10)) & (k < 1000))
    mh_tup = (col == 32 + k % 32)
    mh = jnp.where(pid < 1, mh_comb.astype(jnp.float32),
                   mh_tup.astype(jnp.float32))
    out_ref[0] = jnp.dot(mh, tabs_ref[...],
                         preferred_element_type=jnp.float32)


def _build_tables(tabs):
    return pl.pallas_call(
        _tables_body,
        grid=(2,),
        in_specs=[pl.BlockSpec((64, D), lambda i: (0, 0))],
        out_specs=pl.BlockSpec((1, 1024, D), lambda i: (i, 0, 0)),
        out_shape=jax.ShapeDtypeStruct((2, 1024, D), jnp.float32),
    )(tabs)


def _keys_body(a0_ref, a1_ref, a2_ref, k_ref):
    k_ref[...] = a0_ref[...] * 100 + a1_ref[...] * 10 + a2_ref[...]


def _build_keys(a0, a1, a2):
    return pl.pallas_call(
        _keys_body,
        out_shape=jax.ShapeDtypeStruct(a0.shape, jnp.int32),
    )(a0, a1, a2)


def _x_body(xp_ref, tab_ref, out_ref):
    xv = xp_ref[...]
    col = lax.broadcasted_iota(jnp.int32, (400, 1024), 1)
    mh = jnp.zeros((400, 1024), jnp.float32)
    for c in range(16):
        off = 100 * c if c < 9 else 900
        mh = mh + (col == (xv[:, c:c + 1] + off)).astype(jnp.float32)
    out_ref[...] = jnp.dot(mh, tab_ref[...],
                           preferred_element_type=jnp.float32)


def _x_encode(xp, xt_flat):
    return pl.pallas_call(
        _x_body,
        grid=(N // 400,),
        in_specs=[
            pl.BlockSpec((400, 16), lambda i: (i, 0)),
            pl.BlockSpec((1024, D), lambda i: (0, 0)),
        ],
        out_specs=pl.BlockSpec((400, D), lambda i: (i, 0)),
        out_shape=jax.ShapeDtypeStruct((N, D), jnp.float32),
    )(xp, xt_flat)


_sc_mesh = plsc.VectorSubcoreMesh(core_axis_name="c", subcore_axis_name="s")


@functools.partial(
    pl.kernel,
    mesh=_sc_mesh,
    out_type=[
        jax.ShapeDtypeStruct((E, D), jnp.float32),
        jax.ShapeDtypeStruct((E, D), jnp.float32),
    ],
    scratch_types=[
        pltpu.VMEM((_RPW,), jnp.int32),
        [pltpu.VMEM((_C, D), jnp.float32)] * _NB,
        pltpu.VMEM_SHARED((1024, D), jnp.float32),
        pltpu.VMEM_SHARED((32, D), jnp.float32),
        pltpu.SemaphoreType.DMA,
        pltpu.SemaphoreType.DMA,
    ],
)
def _sc_gather(keys_hbm, comb_hbm, tf_hbm, tt_hbm, ea_out, tup_out,
               idx_v, rows, comb_sh, tup_sh, gsem, osem):
    wid = lax.axis_index("s") * _NC + lax.axis_index("c")
    base0 = wid * _RPW

    @pl.when(lax.axis_index("s") == 0)
    def _():
        pltpu.sync_copy(comb_hbm, comb_sh)
        pltpu.sync_copy(tt_hbm.at[pl.ds(0, 32), :], tup_sh)
    plsc.subcore_barrier()

    def run_job(idx_hbm, table_sh, out_hbm):
        pltpu.sync_copy(idx_hbm.at[pl.ds(base0, _RPW)], idx_v)

        def fire(c, b):
            pltpu.async_copy(
                table_sh.at[idx_v.at[pl.ds(c * _C, _C)]], rows[b], gsem)

        def wait_fire_out(c, b):
            pltpu.make_async_copy(
                table_sh.at[idx_v.at[pl.ds(c * _C, _C)]], rows[b],
                gsem).wait()
            pltpu.async_copy(
                rows[b], out_hbm.at[pl.ds(base0 + c * _C, _C), :], osem)

        def drain_out(b):
            pltpu.make_async_copy(
                rows[b], out_hbm.at[pl.ds(base0, _C), :], osem).wait()

        def group(j, carry):
            for b in range(_NB):
                @pl.when(j > 0)
                def _():
                    drain_out(b)
                fire(j * _NB + b, b)
            for b in range(_NB):
                wait_fire_out(j * _NB + b, b)
            return carry

        lax.fori_loop(0, _NGRP, group, 0)
        for b in range(_NB):
            drain_out(b)
        toff = base0 + _NFULL * _C
        tail_rows = rows[0].at[pl.ds(0, _TAIL), :]
        pltpu.async_copy(
            table_sh.at[idx_v.at[pl.ds(_NFULL * _C, _TAIL)]],
            tail_rows, gsem)
        pltpu.make_async_copy(
            table_sh.at[idx_v.at[pl.ds(_NFULL * _C, _TAIL)]],
            tail_rows, gsem).wait()
        pltpu.sync_copy(tail_rows, out_hbm.at[pl.ds(toff, _TAIL), :])

    run_job(keys_hbm, comb_sh, ea_out)
    run_job(tf_hbm, tup_sh, tup_out)


def kernel(x, A_indices, A_values, tuplefeat, x_tables, ea_tables, tuple_table):
    del A_indices
    tabs = jnp.concatenate([
        ea_tables.reshape(30, D), jnp.zeros((2, D), jnp.float32),
        tuple_table, jnp.zeros((12, D), jnp.float32)], axis=0)
    tables = _build_tables(tabs)
    comb = tables[0]
    tup_pad = tables[1]
    a0 = A_values[:, 0].reshape(E // 512, 512)
    a1 = A_values[:, 1].reshape(E // 512, 512)
    a2 = A_values[:, 2].reshape(E // 512, 512)
    keys = _build_keys(a0, a1, a2)
    ea_emb, tuple_emb = _sc_gather(keys.reshape(E), comb, tuplefeat, tup_pad)
    xp = jnp.pad(x, ((0, 0), (0, 7)))
    xt_flat = jnp.concatenate(
        [x_tables.reshape(900, D), jnp.zeros((124, D), jnp.float32)], axis=0)
    x_emb = _x_encode(xp, xt_flat)
    return (x_emb, ea_emb, tuple_emb)


# speedup vs baseline: 16.7107x; 1.0224x over previous
import functools

import jax
import jax.numpy as jnp
from jax import lax
from jax.experimental import pallas as pl
from jax.experimental.pallas import tpu as pltpu
from jax.experimental.pallas import tpu_sc as plsc

N = 10000
E = 320000
D = 128

_info = plsc.get_sparse_core_info()
_NC, _NS = _info.num_cores, _info.num_subcores
_NW = _NC * _NS
_RPW = E // _NW
_C = 128
_NB = 6
_NFULL = _RPW // _C
_NGRP = _NFULL // _NB
_TAIL = _RPW - _NFULL * _C


def _tables_body(tabs_ref, a0_ref, a1_ref, a2_ref, out_ref, k_ref):
    pid = pl.program_id(0)
    k = lax.broadcasted_iota(jnp.int32, (1024, 1), 0)
    col = lax.broadcasted_iota(jnp.int32, (1024, 64), 1)
    mh_comb = (((col == k // 100) | (col == 10 + (k // 10) % 10)
                | (col == 20 + k % 10)) & (k < 1000))
    mh_tup = (col == 32 + k % 32)
    mh = jnp.where(pid < 1, mh_comb.astype(jnp.float32),
                   mh_tup.astype(jnp.float32))
    out_ref[0] = jnp.dot(mh, tabs_ref[...],
                         preferred_element_type=jnp.float32)

    @pl.when(pid == 0)
    def _():
        k_ref[...] = a0_ref[...] * 100 + a1_ref[...] * 10 + a2_ref[...]


def _build_tables(tabs, a0, a1, a2):
    spec_k = pl.BlockSpec(a0.shape, lambda i: (0, 0))
    return pl.pallas_call(
        _tables_body,
        grid=(2,),
        in_specs=[pl.BlockSpec((64, D), lambda i: (0, 0)),
                  spec_k, spec_k, spec_k],
        out_specs=[pl.BlockSpec((1, 1024, D), lambda i: (i, 0, 0)), spec_k],
        out_shape=[jax.ShapeDtypeStruct((2, 1024, D), jnp.float32),
                   jax.ShapeDtypeStruct(a0.shape, jnp.int32)],
    )(tabs, a0, a1, a2)


_XB = 1000


def _x_body(xp_ref, tab_ref, out_ref):
    xv = xp_ref[...]
    col = lax.broadcasted_iota(jnp.int32, (_XB, 1024), 1)
    mh = jnp.zeros((_XB, 1024), jnp.float32)
    for c in range(16):
        off = 100 * c if c < 9 else 900
        mh = mh + (col == (xv[:, c:c + 1] + off)).astype(jnp.float32)
    out_ref[...] = jnp.dot(mh, tab_ref[...],
                           preferred_element_type=jnp.float32)


def _x_encode(xp, xt_flat):
    return pl.pallas_call(
        _x_body,
        grid=(N // _XB,),
        in_specs=[
            pl.BlockSpec((_XB, 16), lambda i: (i, 0)),
            pl.BlockSpec((1024, D), lambda i: (0, 0)),
        ],
        out_specs=pl.BlockSpec((_XB, D), lambda i: (i, 0)),
        out_shape=jax.ShapeDtypeStruct((N, D), jnp.float32),
    )(xp, xt_flat)


_sc_mesh = plsc.VectorSubcoreMesh(core_axis_name="c", subcore_axis_name="s")


@functools.partial(
    pl.kernel,
    mesh=_sc_mesh,
    out_type=[
        jax.ShapeDtypeStruct((E, D), jnp.float32),
        jax.ShapeDtypeStruct((E, D), jnp.float32),
    ],
    scratch_types=[
        pltpu.VMEM((_RPW,), jnp.int32),
        [pltpu.VMEM((_C, D), jnp.float32)] * _NB,
        pltpu.VMEM_SHARED((1024, D), jnp.float32),
        pltpu.VMEM_SHARED((32, D), jnp.float32),
        pltpu.SemaphoreType.DMA,
        pltpu.SemaphoreType.DMA,
    ],
)
def _sc_gather(keys_hbm, comb_hbm, tf_hbm, tt_hbm, ea_out, tup_out,
               idx_v, rows, comb_sh, tup_sh, gsem, osem):
    wid = lax.axis_index("s") * _NC + lax.axis_index("c")
    base0 = wid * _RPW

    @pl.when(lax.axis_index("s") == 0)
    def _():
        pltpu.sync_copy(comb_hbm, comb_sh)
        pltpu.sync_copy(tt_hbm.at[pl.ds(0, 32), :], tup_sh)
    plsc.subcore_barrier()

    def run_job(idx_hbm, table_sh, out_hbm):
        pltpu.sync_copy(idx_hbm.at[pl.ds(base0, _RPW)], idx_v)

        def fire(c, b):
            pltpu.async_copy(
                table_sh.at[idx_v.at[pl.ds(c * _C, _C)]], rows[b], gsem)

        def wait_fire_out(c, b):
            pltpu.make_async_copy(
                table_sh.at[idx_v.at[pl.ds(c * _C, _C)]], rows[b],
                gsem).wait()
            pltpu.async_copy(
                rows[b], out_hbm.at[pl.ds(base0 + c * _C, _C), :], osem)

        def drain_out(b):
            pltpu.make_async_copy(
                rows[b], out_hbm.at[pl.ds(base0, _C), :], osem).wait()

        def group(j, carry):
            for b in range(_NB):
                @pl.when(j > 0)
                def _():
                    drain_out(b)
                fire(j * _NB + b, b)
            for b in range(_NB):
                wait_fire_out(j * _NB + b, b)
            return carry

        lax.fori_loop(0, _NGRP, group, 0)
        for b in range(_NB):
            drain_out(b)
        toff = base0 + _NFULL * _C
        tail_rows = rows[0].at[pl.ds(0, _TAIL), :]
        pltpu.async_copy(
            table_sh.at[idx_v.at[pl.ds(_NFULL * _C, _TAIL)]],
            tail_rows, gsem)
        pltpu.make_async_copy(
            table_sh.at[idx_v.at[pl.ds(_NFULL * _C, _TAIL)]],
            tail_rows, gsem).wait()
        pltpu.sync_copy(tail_rows, out_hbm.at[pl.ds(toff, _TAIL), :])

    run_job(keys_hbm, comb_sh, ea_out)
    run_job(tf_hbm, tup_sh, tup_out)


def kernel(x, A_indices, A_values, tuplefeat, x_tables, ea_tables, tuple_table):
    del A_indices
    tabs = jnp.concatenate([
        ea_tables.reshape(30, D), jnp.zeros((2, D), jnp.float32),
        tuple_table, jnp.zeros((12, D), jnp.float32)], axis=0)
    a0 = A_values[:, 0].reshape(E // 512, 512)
    a1 = A_values[:, 1].reshape(E // 512, 512)
    a2 = A_values[:, 2].reshape(E // 512, 512)
    tables, keys = _build_tables(tabs, a0, a1, a2)
    comb = tables[0]
    tup_pad = tables[1]
    ea_emb, tuple_emb = _sc_gather(keys.reshape(E), comb, tuplefeat, tup_pad)
    xp = jnp.pad(x, ((0, 0), (0, 7)))
    xt_flat = jnp.concatenate(
        [x_tables.reshape(900, D), jnp.zeros((124, D), jnp.float32)], axis=0)
    x_emb = _x_encode(xp, xt_flat)
    return (x_emb, ea_emb, tuple_emb)


# speedup vs baseline: 16.8388x; 1.0077x over previous
import functools

import jax
import jax.numpy as jnp
from jax import lax
from jax.experimental import pallas as pl
from jax.experimental.pallas import tpu as pltpu
from jax.experimental.pallas import tpu_sc as plsc

N = 10000
E = 320000
D = 128

_info = plsc.get_sparse_core_info()
_NC, _NS = _info.num_cores, _info.num_subcores
_NW = _NC * _NS
_RPW = E // _NW
_C = 128
_NB = 6
_NFULL = _RPW // _C
_NGRP = _NFULL // _NB
_TAIL = _RPW - _NFULL * _C


def _tables_body(tabs_ref, a0_ref, a1_ref, a2_ref, out_ref, k_ref):
    pid = pl.program_id(0)
    k = lax.broadcasted_iota(jnp.int32, (1024, 1), 0)
    col = lax.broadcasted_iota(jnp.int32, (1024, 64), 1)
    mh_comb = (((col == k // 100) | (col == 10 + (k // 10) % 10)
                | (col == 20 + k % 10)) & (k < 1000))
    mh_tup = (col == 32 + k % 32)
    mh = jnp.where(pid < 1, mh_comb.astype(jnp.float32),
                   mh_tup.astype(jnp.float32))
    out_ref[0] = jnp.dot(mh, tabs_ref[...],
                         preferred_element_type=jnp.float32)

    @pl.when(pid == 0)
    def _():
        k_ref[...] = a0_ref[...] * 100 + a1_ref[...] * 10 + a2_ref[...]


def _build_tables(tabs, a0, a1, a2):
    spec_k = pl.BlockSpec(a0.shape, lambda i: (0, 0))
    return pl.pallas_call(
        _tables_body,
        grid=(2,),
        in_specs=[pl.BlockSpec((64, D), lambda i: (0, 0)),
                  spec_k, spec_k, spec_k],
        out_specs=[pl.BlockSpec((1, 1024, D), lambda i: (i, 0, 0)), spec_k],
        out_shape=[jax.ShapeDtypeStruct((2, 1024, D), jnp.float32),
                   jax.ShapeDtypeStruct(a0.shape, jnp.int32)],
    )(tabs, a0, a1, a2)


_XB = 1000


def _x_body(xp_ref, tab_ref, out_ref):
    xv = xp_ref[...]
    col = lax.broadcasted_iota(jnp.int32, (_XB, 1024), 1)
    mh = jnp.zeros((_XB, 1024), jnp.float32)
    for c in range(9):
        mh = mh + (col == (xv[:, c:c + 1] + 100 * c)).astype(jnp.float32)
    out_ref[...] = jnp.dot(mh, tab_ref[...],
                           preferred_element_type=jnp.float32)


def _x_encode(xp, xt_flat):
    return pl.pallas_call(
        _x_body,
        grid=(N // _XB,),
        in_specs=[
            pl.BlockSpec((_XB, 9), lambda i: (i, 0)),
            pl.BlockSpec((1024, D), lambda i: (0, 0)),
        ],
        out_specs=pl.BlockSpec((_XB, D), lambda i: (i, 0)),
        out_shape=jax.ShapeDtypeStruct((N, D), jnp.float32),
    )(xp, xt_flat)


_sc_mesh = plsc.VectorSubcoreMesh(core_axis_name="c", subcore_axis_name="s")


@functools.partial(
    pl.kernel,
    mesh=_sc_mesh,
    out_type=[
        jax.ShapeDtypeStruct((E, D), jnp.float32),
        jax.ShapeDtypeStruct((E, D), jnp.float32),
    ],
    scratch_types=[
        [pltpu.VMEM((_RPW,), jnp.int32)] * 2,
        [pltpu.VMEM((_C, D), jnp.float32)] * _NB,
        pltpu.VMEM_SHARED((1024, D), jnp.float32),
        pltpu.VMEM_SHARED((32, D), jnp.float32),
        pltpu.SemaphoreType.DMA,
        pltpu.SemaphoreType.DMA,
        pltpu.SemaphoreType.DMA,
    ],
)
def _sc_gather(keys_hbm, comb_hbm, tf_hbm, tt_hbm, ea_out, tup_out,
               idxs, rows, comb_sh, tup_sh, gsem, osem, isem):
    wid = lax.axis_index("s") * _NC + lax.axis_index("c")
    base0 = wid * _RPW

    pltpu.async_copy(keys_hbm.at[pl.ds(base0, _RPW)], idxs[0], isem)
    pltpu.async_copy(tf_hbm.at[pl.ds(base0, _RPW)], idxs[1], isem)
    @pl.when(lax.axis_index("s") == 0)
    def _():
        pltpu.sync_copy(comb_hbm, comb_sh)
        pltpu.sync_copy(tt_hbm.at[pl.ds(0, 32), :], tup_sh)
    pltpu.make_async_copy(
        keys_hbm.at[pl.ds(base0, _RPW)], idxs[0], isem).wait()
    pltpu.make_async_copy(
        tf_hbm.at[pl.ds(base0, _RPW)], idxs[1], isem).wait()
    plsc.subcore_barrier()

    def run_job(idx_v, table_sh, out_hbm):
        def fire(c, b):
            pltpu.async_copy(
                table_sh.at[idx_v.at[pl.ds(c * _C, _C)]], rows[b], gsem)

        def wait_fire_out(c, b):
            pltpu.make_async_copy(
                table_sh.at[idx_v.at[pl.ds(c * _C, _C)]], rows[b],
                gsem).wait()
            pltpu.async_copy(
                rows[b], out_hbm.at[pl.ds(base0 + c * _C, _C), :], osem)

        def drain_out(b):
            pltpu.make_async_copy(
                rows[b], out_hbm.at[pl.ds(base0, _C), :], osem).wait()

        def group(j, carry):
            for b in range(_NB):
                @pl.when(j > 0)
                def _():
                    drain_out(b)
                fire(j * _NB + b, b)
            for b in range(_NB):
                wait_fire_out(j * _NB + b, b)
            return carry

        lax.fori_loop(0, _NGRP, group, 0)
        for b in range(_NB):
            drain_out(b)
        toff = base0 + _NFULL * _C
        tail_rows = rows[0].at[pl.ds(0, _TAIL), :]
        pltpu.async_copy(
            table_sh.at[idx_v.at[pl.ds(_NFULL * _C, _TAIL)]],
            tail_rows, gsem)
        pltpu.make_async_copy(
            table_sh.at[idx_v.at[pl.ds(_NFULL * _C, _TAIL)]],
            tail_rows, gsem).wait()
        pltpu.sync_copy(tail_rows, out_hbm.at[pl.ds(toff, _TAIL), :])

    run_job(idxs[0], comb_sh, ea_out)
    run_job(idxs[1], tup_sh, tup_out)


def kernel(x, A_indices, A_values, tuplefeat, x_tables, ea_tables, tuple_table):
    del A_indices
    tabs = jnp.concatenate([
        ea_tables.reshape(30, D), jnp.zeros((2, D), jnp.float32),
        tuple_table, jnp.zeros((12, D), jnp.float32)], axis=0)
    a0 = A_values[:, 0].reshape(E // 512, 512)
    a1 = A_values[:, 1].reshape(E // 512, 512)
    a2 = A_values[:, 2].reshape(E // 512, 512)
    tables, keys = _build_tables(tabs, a0, a1, a2)
    comb = tables[0]
    tup_pad = tables[1]
    ea_emb, tuple_emb = _sc_gather(keys.reshape(E), comb, tuplefeat, tup_pad)
    xt_flat = jnp.concatenate(
        [x_tables.reshape(900, D), jnp.zeros((124, D), jnp.float32)], axis=0)
    x_emb = _x_encode(x, xt_flat)
    return (x_emb, ea_emb, tuple_emb)
